# Initial kernel scaffold; baseline (speedup 1.0000x reference)
import jax
import jax.numpy as jnp
from jax.experimental import pallas as pl


def kernel(x, edge_index, edge_attr, Wl1, bl1, Wr1, Wl2, bl2, Wr2, g1, be1, g2, be2, W1, b1, W2, b2, W3, b3):
    raise NotImplementedError("write your pallas kernel here")


# speedup vs baseline: 3.5026x; 3.5026x over previous
import functools

import jax
import jax.numpy as jnp
from jax import lax
from jax.experimental import pallas as pl
from jax.experimental.pallas import tpu as pltpu
from jax.experimental.pallas import tpu_sc as plsc

_N = 10000
_E = 320000
_D = 128
_DE = 16
_H = 128

_NC = 2
_NS = 16
_NW = _NC * _NS
_EPW = _E // _NW
_W = 80
_NWIN = _EPW // _W
_ZR = 80
_NCH = _N // _ZR
_CW = 16


def _mesh():
    return plsc.VectorSubcoreMesh(core_axis_name="c", subcore_axis_name="s")


def _fill_2d(ref, rows, cols, value):
    vec = jnp.full((16,), value, jnp.float32)

    @pl.loop(0, rows)
    def _(r):
        @pl.loop(0, cols, step=16)
        def _(j):
            ref[r, pl.ds(j, 16)] = vec


def _sc_segsum(table, src2d, dst2d):
    out_type = jax.ShapeDtypeStruct((_NC * _N, _D), jnp.float32)
    scratch = [
        pltpu.VMEM((_NWIN, _W), jnp.int32),
        pltpu.VMEM((_NWIN, _W), jnp.int32),
        pltpu.VMEM((_W, _D), jnp.float32),
        pltpu.VMEM_SHARED((_N, _D), jnp.float32),
        pltpu.SemaphoreType.DMA,
    ]

    @functools.partial(
        pl.kernel, mesh=_mesh(), out_type=out_type, scratch_types=scratch)
    def k(table_hbm, src_hbm, dst_hbm, sum_out, srcv, dstv, rows, acc, sem):
        c = lax.axis_index("c")
        s = lax.axis_index("s")
        wid = c * _NS + s

        _fill_2d(rows, _ZR, _D, 0.0)

        @pl.loop(s, _NCH, step=_NS)
        def _(k):
            pltpu.sync_copy(rows, acc.at[pl.ds(k * _ZR, _ZR)])

        pltpu.sync_copy(src_hbm.at[wid], srcv)
        pltpu.sync_copy(dst_hbm.at[wid], dstv)
        plsc.subcore_barrier()

        @pl.loop(0, _NWIN)
        def _(w):
            pltpu.async_copy(table_hbm.at[srcv.at[w]], rows, sem).wait()
            pltpu.sync_copy(rows, acc.at[dstv.at[w]], add=True)

        plsc.subcore_barrier()
        @pl.loop(s, _NCH, step=_NS)
        def _(k):
            pltpu.sync_copy(acc.at[pl.ds(k * _ZR, _ZR)],
                            sum_out.at[pl.ds(c * _N + k * _ZR, _ZR)])

    return k(table, src2d, dst2d)


def _sc_counts(dst2d):
    out_type = jax.ShapeDtypeStruct((_NC * _N, _D), jnp.float32)
    scratch = [
        pltpu.VMEM((_NWIN, _W), jnp.int32),
        pltpu.VMEM((_W, _D), jnp.float32),
        pltpu.VMEM_SHARED((_N, _D), jnp.float32),
    ]

    @functools.partial(
        pl.kernel, mesh=_mesh(), out_type=out_type, scratch_types=scratch)
    def k(dst_hbm, cnt_out, dstv, ones, cacc):
        c = lax.axis_index("c")
        s = lax.axis_index("s")
        wid = c * _NS + s

        _fill_2d(ones, _W, _D, 0.0)

        @pl.loop(s, _NCH, step=_NS)
        def _(k):
            pltpu.sync_copy(ones, cacc.at[pl.ds(k * _ZR, _ZR)])

        _fill_2d(ones, _W, _D, 1.0)
        pltpu.sync_copy(dst_hbm.at[wid], dstv)
        plsc.subcore_barrier()

        @pl.loop(0, _NWIN)
        def _(w):
            pltpu.sync_copy(ones, cacc.at[dstv.at[w]], add=True)

        plsc.subcore_barrier()

        @pl.loop(s, _NCH, step=_NS)
        def _(k):
            pltpu.sync_copy(cacc.at[pl.ds(k * _ZR, _ZR)],
                            cnt_out.at[pl.ds(c * _N + k * _ZR, _ZR)])

    return k(dst2d)


def _sc_edge_gather(z, src2d, dst2d):
    out_type = [jax.ShapeDtypeStruct((_E, _D), jnp.float32),
                jax.ShapeDtypeStruct((_E, _D), jnp.float32)]
    scratch = [
        pltpu.VMEM((_NWIN, _W), jnp.int32),
        pltpu.VMEM((_NWIN, _W), jnp.int32),
        pltpu.VMEM((_W, _D), jnp.float32),
        pltpu.VMEM((_W, _D), jnp.float32),
        pltpu.SemaphoreType.DMA,
        pltpu.SemaphoreType.DMA,
    ]

    @functools.partial(
        pl.kernel, mesh=_mesh(), out_type=out_type, scratch_types=scratch)
    def k(z_hbm, src_hbm, dst_hbm, zs_out, zd_out, srcv, dstv, ra, rb,
          sema, semb):
        c = lax.axis_index("c")
        s = lax.axis_index("s")
        wid = c * _NS + s
        pltpu.sync_copy(src_hbm.at[wid], srcv)
        pltpu.sync_copy(dst_hbm.at[wid], dstv)

        @pl.loop(0, _NWIN)
        def _(w):
            base = wid * _EPW + w * _W
            pltpu.async_copy(z_hbm.at[srcv.at[w]], ra, sema).wait()
            pltpu.sync_copy(ra, zs_out.at[pl.ds(base, _W)])
            pltpu.async_copy(z_hbm.at[dstv.at[w]], rb, semb).wait()
            pltpu.sync_copy(rb, zd_out.at[pl.ds(base, _W)])

    return k(z, src2d, dst2d)


_BN = 1000


def _tc_node_body(p_ref, c_ref, x_ref, wl_ref, wr_ref, v_ref, o_ref):
    psum = p_ref[0] + p_ref[1]
    cnt = c_ref[0][:, :1] + c_ref[1][:, :1]
    mean = psum / jnp.maximum(cnt, 1.0)
    pre = (jnp.dot(mean, wl_ref[...], preferred_element_type=jnp.float32)
           + jnp.dot(x_ref[...], wr_ref[...],
                     preferred_element_type=jnp.float32)
           + v_ref[0:1, :])
    mu = jnp.mean(pre, axis=-1, keepdims=True)
    var = jnp.mean((pre - mu) ** 2, axis=-1, keepdims=True)
    yn = (pre - mu) * lax.rsqrt(var + 1e-5)
    o_ref[...] = jnp.maximum(yn * v_ref[1:2, :] + v_ref[2:3, :], 0.0)


def _tc_node(parts, cnts, xin, Wl, bl, Wr, g, be):
    parts = parts.reshape(_NC, _N, _D)
    cnts = cnts.reshape(_NC, _N, _D)
    v = jnp.stack([bl, g, be])
    return pl.pallas_call(
        _tc_node_body,
        grid=(_N // _BN,),
        in_specs=[
            pl.BlockSpec((_NC, _BN, _D), lambda i: (0, i, 0)),
            pl.BlockSpec((_NC, _BN, _D), lambda i: (0, i, 0)),
            pl.BlockSpec((_BN, _D), lambda i: (i, 0)),
            pl.BlockSpec((_D, _H), lambda i: (0, 0)),
            pl.BlockSpec((_D, _H), lambda i: (0, 0)),
            pl.BlockSpec((3, _H), lambda i: (0, 0)),
        ],
        out_specs=pl.BlockSpec((_BN, _H), lambda i: (i, 0)),
        out_shape=jax.ShapeDtypeStruct((_N, _H), jnp.float32),
    )(parts, cnts, xin, Wl.T, Wr.T, v)


_BE = 1000


def _tc_edge_body(zs_ref, zd_ref, ea_ref, w1s_ref, w1d_ref, w1p_ref,
                  w1e_ref, w2_ref, v_ref, o_ref):
    zs = zs_ref[...]
    zd = zd_ref[...]
    t = (jnp.dot(zs, w1s_ref[...], preferred_element_type=jnp.float32)
         + jnp.dot(zd, w1d_ref[...], preferred_element_type=jnp.float32)
         + jnp.dot(zs * zd, w1p_ref[...], preferred_element_type=jnp.float32)
         + jnp.dot(ea_ref[...], w1e_ref[...],
                   preferred_element_type=jnp.float32)
         + v_ref[0:1, :])
    h1 = jnp.maximum(t, 0.0)
    h2 = jnp.maximum(
        jnp.dot(h1, w2_ref[...], preferred_element_type=jnp.float32)
        + v_ref[1:2, :], 0.0)
    o_ref[...] = (jnp.sum(h2 * v_ref[2:3, :], axis=-1, keepdims=True)
                  + v_ref[3:4, 0:1])


def _tc_edge_mlp(zs, zd, ea, W1, b1, W2, b2, W3, b3):
    w1s = W1[:, :_H].T
    w1d = W1[:, _H:2 * _H].T
    w1p = W1[:, 2 * _H:3 * _H].T
    w1e = W1[:, 3 * _H:].T
    w2p = jnp.pad(W2.T, ((0, 0), (0, _H - W2.shape[0])))
    b2p = jnp.pad(b2, (0, _H - b2.shape[0]))
    w3p = jnp.pad(W3[0], (0, _H - W3.shape[1]))
    v = jnp.stack([b1, b2p, w3p, jnp.broadcast_to(b3, (_H,))])
    out = pl.pallas_call(
        _tc_edge_body,
        grid=(_E // _BE,),
        in_specs=[
            pl.BlockSpec((_BE, _D), lambda i: (i, 0)),
            pl.BlockSpec((_BE, _D), lambda i: (i, 0)),
            pl.BlockSpec((_BE, _DE), lambda i: (i, 0)),
            pl.BlockSpec((_D, _H), lambda i: (0, 0)),
            pl.BlockSpec((_D, _H), lambda i: (0, 0)),
            pl.BlockSpec((_D, _H), lambda i: (0, 0)),
            pl.BlockSpec((_DE, _H), lambda i: (0, 0)),
            pl.BlockSpec((_H, _H), lambda i: (0, 0)),
            pl.BlockSpec((4, _H), lambda i: (0, 0)),
        ],
        out_specs=pl.BlockSpec((_BE, 1), lambda i: (i, 0)),
        out_shape=jax.ShapeDtypeStruct((_E, 1), jnp.float32),
    )(zs, zd, ea, w1s, w1d, w1p, w1e, w2p, v)
    return out[:, 0]


def kernel(x, edge_index, edge_attr, Wl1, bl1, Wr1, Wl2, bl2, Wr2, g1, be1,
           g2, be2, W1, b1, W2, b2, W3, b3):
    src2d = edge_index[0].reshape(_NW, _NWIN, _W)
    dst2d = edge_index[1].reshape(_NW, _NWIN, _W)

    cnts = _sc_counts(dst2d)
    sums1 = _sc_segsum(x, src2d, dst2d)
    h = _tc_node(sums1, cnts, x, Wl1, bl1, Wr1, g1, be1)
    sums2 = _sc_segsum(h, src2d, dst2d)
    z = _tc_node(sums2, cnts, h, Wl2, bl2, Wr2, g2, be2)
    zs, zd = _sc_edge_gather(z, src2d, dst2d)
    return _tc_edge_mlp(zs, zd, edge_attr, W1, b1, W2, b2, W3, b3)


# speedup vs baseline: 3.8329x; 1.0943x over previous
import functools

import jax
import jax.numpy as jnp
from jax import lax
from jax.experimental import pallas as pl
from jax.experimental.pallas import tpu as pltpu
from jax.experimental.pallas import tpu_sc as plsc

_N = 10000
_E = 320000
_D = 128
_DE = 16
_H = 128

_NC = 2
_NS = 16
_NW = _NC * _NS
_EPW = _E // _NW
_W = 80
_NWIN = _EPW // _W
_ZR = 80
_NCH = _N // _ZR
_CW = 16


def _mesh():
    return plsc.VectorSubcoreMesh(core_axis_name="c", subcore_axis_name="s")


def _fill_2d(ref, rows, cols, value):
    vec = jnp.full((16,), value, jnp.float32)

    @pl.loop(0, rows)
    def _(r):
        @pl.loop(0, cols, step=16)
        def _(j):
            ref[r, pl.ds(j, 16)] = vec


_CHW = 40


def _sc_segsum(table, srcpad, dst2d):
    out_type = jax.ShapeDtypeStruct((_NC * _N, _D), jnp.float32)
    scratch = [
        pltpu.VMEM((_NWIN, _W), jnp.int32),
        pltpu.VMEM((2, _CHW, _W), jnp.int32),
        pltpu.VMEM((_W, _D), jnp.float32),
        pltpu.VMEM((_W, _D), jnp.float32),
        pltpu.VMEM_SHARED((_N, _D), jnp.float32),
        pltpu.SemaphoreType.DMA,
        pltpu.SemaphoreType.DMA,
    ]

    @functools.partial(
        pl.kernel, mesh=_mesh(), out_type=out_type, scratch_types=scratch)
    def k(table_hbm, src_hbm, dst_hbm, sum_out, dstv, srcc, ra, rb, acc,
          sema, semb):
        c = lax.axis_index("c")
        s = lax.axis_index("s")
        wid = c * _NS + s

        def src_row(j):
            return srcc.at[lax.rem(j // _CHW, 2), lax.rem(j, _CHW)]

        def gather(j, buf, sem):
            pltpu.async_copy(table_hbm.at[src_row(j)], buf, sem)

        def gwait(buf, sem):
            pltpu.make_async_copy(table_hbm.at[srcc.at[0, 0]], buf, sem
                                  ).wait()

        _fill_2d(ra, _ZR, _D, 0.0)

        @pl.loop(s, _NCH, step=_NS)
        def _(k):
            pltpu.sync_copy(ra, acc.at[pl.ds(k * _ZR, _ZR)])

        pltpu.sync_copy(dst_hbm.at[wid], dstv)
        pltpu.sync_copy(src_hbm.at[wid, pl.ds(0, _CHW)], srcc.at[0])
        plsc.subcore_barrier()

        gather(0, ra, sema)
        gather(1, rb, semb)

        @pl.loop(0, _NWIN - 1, step=2)
        def _(w):
            @pl.when(lax.rem(w + 2, _CHW) == 0)
            def _():
                pltpu.sync_copy(
                    src_hbm.at[wid, pl.ds(pl.multiple_of(w + 2, _CHW), _CHW)],
                    srcc.at[lax.rem((w + 2) // _CHW, 2)])

            gwait(ra, sema)
            pltpu.sync_copy(ra, acc.at[dstv.at[w]], add=True)
            gather(w + 2, ra, sema)
            gwait(rb, semb)
            pltpu.sync_copy(rb, acc.at[dstv.at[w + 1]], add=True)
            gather(w + 3, rb, semb)

        gwait(ra, sema)
        pltpu.sync_copy(ra, acc.at[dstv.at[_NWIN - 1]], add=True)
        gwait(rb, semb)

        plsc.subcore_barrier()
        @pl.loop(s, _NCH, step=_NS)
        def _(k):
            pltpu.sync_copy(acc.at[pl.ds(k * _ZR, _ZR)],
                            sum_out.at[pl.ds(c * _N + k * _ZR, _ZR)])

    return k(table, srcpad, dst2d)


def _sc_counts(dst2d):
    out_type = jax.ShapeDtypeStruct((_NC * _N, _D), jnp.float32)
    scratch = [
        pltpu.VMEM((_NWIN, _W), jnp.int32),
        pltpu.VMEM((_W, _D), jnp.float32),
        pltpu.VMEM_SHARED((_N, _D), jnp.float32),
    ]

    @functools.partial(
        pl.kernel, mesh=_mesh(), out_type=out_type, scratch_types=scratch)
    def k(dst_hbm, cnt_out, dstv, ones, cacc):
        c = lax.axis_index("c")
        s = lax.axis_index("s")
        wid = c * _NS + s

        _fill_2d(ones, _W, _D, 0.0)

        @pl.loop(s, _NCH, step=_NS)
        def _(k):
            pltpu.sync_copy(ones, cacc.at[pl.ds(k * _ZR, _ZR)])

        _fill_2d(ones, _W, _D, 1.0)
        pltpu.sync_copy(dst_hbm.at[wid], dstv)
        plsc.subcore_barrier()

        @pl.loop(0, _NWIN)
        def _(w):
            pltpu.sync_copy(ones, cacc.at[dstv.at[w]], add=True)

        plsc.subcore_barrier()

        @pl.loop(s, _NCH, step=_NS)
        def _(k):
            pltpu.sync_copy(cacc.at[pl.ds(k * _ZR, _ZR)],
                            cnt_out.at[pl.ds(c * _N + k * _ZR, _ZR)])

    return k(dst2d)


def _sc_edge_gather(z, src2d, dst2d):
    out_type = [jax.ShapeDtypeStruct((_E, _D), jnp.float32),
                jax.ShapeDtypeStruct((_E, _D), jnp.float32)]
    scratch = [
        pltpu.VMEM((_NWIN, _W), jnp.int32),
        pltpu.VMEM((_NWIN, _W), jnp.int32),
        pltpu.VMEM((_W, _D), jnp.float32),
        pltpu.VMEM((_W, _D), jnp.float32),
        pltpu.SemaphoreType.DMA,
        pltpu.SemaphoreType.DMA,
    ]

    scratch += [
        pltpu.VMEM((_W, _D), jnp.float32),
        pltpu.VMEM((_W, _D), jnp.float32),
        pltpu.SemaphoreType.DMA,
        pltpu.SemaphoreType.DMA,
    ]

    @functools.partial(
        pl.kernel, mesh=_mesh(), out_type=out_type, scratch_types=scratch)
    def k(z_hbm, src_hbm, dst_hbm, zs_out, zd_out, srcv, dstv, sa, da,
          sema, semda, sb, db, semb, semdb):
        c = lax.axis_index("c")
        s = lax.axis_index("s")
        wid = c * _NS + s
        pltpu.sync_copy(src_hbm.at[wid], srcv)
        pltpu.sync_copy(dst_hbm.at[wid], dstv)

        def gwait(buf, sem):
            pltpu.make_async_copy(z_hbm.at[srcv.at[0]], buf, sem).wait()

        pltpu.async_copy(z_hbm.at[srcv.at[0]], sa, sema)
        pltpu.async_copy(z_hbm.at[dstv.at[0]], da, semda)
        pltpu.async_copy(z_hbm.at[srcv.at[1]], sb, semb)
        pltpu.async_copy(z_hbm.at[dstv.at[1]], db, semdb)

        @pl.loop(0, _NWIN - 1, step=2)
        def _(w):
            base = wid * _EPW + w * _W
            gwait(sa, sema)
            pltpu.sync_copy(sa, zs_out.at[pl.ds(base, _W)])
            pltpu.async_copy(z_hbm.at[srcv.at[w + 2]], sa, sema)
            gwait(da, semda)
            pltpu.sync_copy(da, zd_out.at[pl.ds(base, _W)])
            pltpu.async_copy(z_hbm.at[dstv.at[w + 2]], da, semda)
            gwait(sb, semb)
            pltpu.sync_copy(sb, zs_out.at[pl.ds(base + _W, _W)])
            gwait(db, semdb)
            pltpu.sync_copy(db, zd_out.at[pl.ds(base + _W, _W)])

            @pl.when(w + 3 < _NWIN)
            def _():
                pltpu.async_copy(z_hbm.at[srcv.at[w + 3]], sb, semb)
                pltpu.async_copy(z_hbm.at[dstv.at[w + 3]], db, semdb)

        base = wid * _EPW + (_NWIN - 1) * _W
        gwait(sa, sema)
        pltpu.sync_copy(sa, zs_out.at[pl.ds(base, _W)])
        gwait(da, semda)
        pltpu.sync_copy(da, zd_out.at[pl.ds(base, _W)])

    return k(z, src2d, dst2d)


_BN = 1000


def _tc_node_body(p_ref, c_ref, x_ref, wl_ref, wr_ref, v_ref, o_ref):
    psum = p_ref[0] + p_ref[1]
    cnt = c_ref[0][:, :1] + c_ref[1][:, :1]
    mean = psum / jnp.maximum(cnt, 1.0)
    pre = (jnp.dot(mean, wl_ref[...], preferred_element_type=jnp.float32)
           + jnp.dot(x_ref[...], wr_ref[...],
                     preferred_element_type=jnp.float32)
           + v_ref[0:1, :])
    mu = jnp.mean(pre, axis=-1, keepdims=True)
    var = jnp.mean((pre - mu) ** 2, axis=-1, keepdims=True)
    yn = (pre - mu) * lax.rsqrt(var + 1e-5)
    o_ref[...] = jnp.maximum(yn * v_ref[1:2, :] + v_ref[2:3, :], 0.0)


def _tc_node(parts, cnts, xin, Wl, bl, Wr, g, be):
    parts = parts.reshape(_NC, _N, _D)
    cnts = cnts.reshape(_NC, _N, _D)
    v = jnp.stack([bl, g, be])
    return pl.pallas_call(
        _tc_node_body,
        grid=(_N // _BN,),
        in_specs=[
            pl.BlockSpec((_NC, _BN, _D), lambda i: (0, i, 0)),
            pl.BlockSpec((_NC, _BN, _D), lambda i: (0, i, 0)),
            pl.BlockSpec((_BN, _D), lambda i: (i, 0)),
            pl.BlockSpec((_D, _H), lambda i: (0, 0)),
            pl.BlockSpec((_D, _H), lambda i: (0, 0)),
            pl.BlockSpec((3, _H), lambda i: (0, 0)),
        ],
        out_specs=pl.BlockSpec((_BN, _H), lambda i: (i, 0)),
        out_shape=jax.ShapeDtypeStruct((_N, _H), jnp.float32),
    )(parts, cnts, xin, Wl.T, Wr.T, v)


_BE = 1000


def _tc_edge_body(zs_ref, zd_ref, ea_ref, w1s_ref, w1d_ref, w1p_ref,
                  w1e_ref, w2_ref, v_ref, o_ref):
    zs = zs_ref[...]
    zd = zd_ref[...]
    bf = jnp.bfloat16
    t = (jnp.dot(zs.astype(bf), w1s_ref[...],
                 preferred_element_type=jnp.float32)
         + jnp.dot(zd.astype(bf), w1d_ref[...],
                   preferred_element_type=jnp.float32)
         + jnp.dot((zs * zd).astype(bf), w1p_ref[...],
                   preferred_element_type=jnp.float32)
         + jnp.dot(ea_ref[...].astype(bf), w1e_ref[...],
                   preferred_element_type=jnp.float32)
         + v_ref[0:1, :])
    h1 = jnp.maximum(t, 0.0)
    h2 = jnp.maximum(
        jnp.dot(h1.astype(bf), w2_ref[...], preferred_element_type=jnp.float32)
        + v_ref[1:2, :], 0.0)
    o_ref[...] = (jnp.sum(h2 * v_ref[2:3, :], axis=-1, keepdims=True)
                  + v_ref[3:4, 0:1])


def _tc_edge_mlp(zs, zd, ea, W1, b1, W2, b2, W3, b3):
    bf = jnp.bfloat16
    w1s = W1[:, :_H].T.astype(bf)
    w1d = W1[:, _H:2 * _H].T.astype(bf)
    w1p = W1[:, 2 * _H:3 * _H].T.astype(bf)
    w1e = W1[:, 3 * _H:].T.astype(bf)
    w2p = jnp.pad(W2.T, ((0, 0), (0, _H - W2.shape[0]))).astype(bf)
    b2p = jnp.pad(b2, (0, _H - b2.shape[0]))
    w3p = jnp.pad(W3[0], (0, _H - W3.shape[1]))
    v = jnp.stack([b1, b2p, w3p, jnp.broadcast_to(b3, (_H,))])
    out = pl.pallas_call(
        _tc_edge_body,
        grid=(_E // _BE,),
        in_specs=[
            pl.BlockSpec((_BE, _D), lambda i: (i, 0)),
            pl.BlockSpec((_BE, _D), lambda i: (i, 0)),
            pl.BlockSpec((_BE, _DE), lambda i: (i, 0)),
            pl.BlockSpec((_D, _H), lambda i: (0, 0)),
            pl.BlockSpec((_D, _H), lambda i: (0, 0)),
            pl.BlockSpec((_D, _H), lambda i: (0, 0)),
            pl.BlockSpec((_DE, _H), lambda i: (0, 0)),
            pl.BlockSpec((_H, _H), lambda i: (0, 0)),
            pl.BlockSpec((4, _H), lambda i: (0, 0)),
        ],
        out_specs=pl.BlockSpec((_BE, 1), lambda i: (i, 0)),
        out_shape=jax.ShapeDtypeStruct((_E, 1), jnp.float32),
    )(zs, zd, ea, w1s, w1d, w1p, w1e, w2p, v)
    return out[:, 0]


def kernel(x, edge_index, edge_attr, Wl1, bl1, Wr1, Wl2, bl2, Wr2, g1, be1,
           g2, be2, W1, b1, W2, b2, W3, b3):
    src2d = edge_index[0].reshape(_NW, _NWIN, _W)
    dst2d = edge_index[1].reshape(_NW, _NWIN, _W)
    srcpad = jnp.pad(src2d, ((0, 0), (0, 4 * _CHW - _NWIN), (0, 0)))

    cnts = _sc_counts(dst2d)
    sums1 = _sc_segsum(x, srcpad, dst2d)
    h = _tc_node(sums1, cnts, x, Wl1, bl1, Wr1, g1, be1)
    sums2 = _sc_segsum(h, srcpad, dst2d)
    z = _tc_node(sums2, cnts, h, Wl2, bl2, Wr2, g2, be2)
    zs, zd = _sc_edge_gather(z, src2d, dst2d)
    return _tc_edge_mlp(zs, zd, edge_attr, W1, b1, W2, b2, W3, b3)
